# Initial kernel scaffold; baseline (speedup 1.0000x reference)
import functools

import jax
import jax.numpy as jnp
from jax import lax
from jax.experimental import pallas as pl
from jax.experimental.pallas import tpu as pltpu
from jax.experimental.pallas import tpu_sc as plsc

N = 10000
B = 64
K = 8
H = 16
NB = 512
NTOT = N + NB


def _elu(x):
    return jnp.where(x > 0, x, jnp.exp(x) - 1.0)


def _tc_encoder_body(x_ref, W1_ref, b1_ref, W2_ref, b2_ref, Wd_ref, bc_ref,
                     Wcb_ref, h_ref, C1_ref, Bm1_ref):
    x = x_ref[...]
    h1 = _elu(jnp.dot(x, W1_ref[...], preferred_element_type=jnp.float32)
              + b1_ref[...])
    h = _elu(jnp.dot(h1, W2_ref[...], preferred_element_type=jnp.float32)
             + b2_ref[...])
    h_ref[...] = h
    C1_ref[...] = (jnp.dot(h, Wd_ref[...], preferred_element_type=jnp.float32)
                   + bc_ref[...])
    Bm1_ref[...] = jnp.dot(h, Wcb_ref[...], preferred_element_type=jnp.float32)


def _tc_encoder(x, W1, b1, W2, b2, Wd, bc, Wcb):
    R = 1000
    grid = (N // R,)
    full = lambda shape: pl.BlockSpec(shape, lambda i: (0, 0))
    return pl.pallas_call(
        _tc_encoder_body,
        grid=grid,
        in_specs=[
            pl.BlockSpec((R, 4), lambda i: (i, 0)),
            full((4, H)), full((1, H)), full((H, H)), full((1, H)),
            full((H, H)), full((1, H)), full((H, H)),
        ],
        out_specs=[
            pl.BlockSpec((R, H), lambda i: (i, 0)),
            pl.BlockSpec((R, H), lambda i: (i, 0)),
            pl.BlockSpec((R, H), lambda i: (i, 0)),
        ],
        out_shape=[jax.ShapeDtypeStruct((N, H), jnp.float32)] * 3,
    )(x, W1, b1, W2, b2, Wd, bc, Wcb)


def _seg_sq(hT_v, sq_v, g_hi):
    def body(g, _):
        base = g * 16
        v0 = hT_v[0, pl.ds(base, 16)]
        a = v0 * v0
        for f in range(1, H):
            v = hT_v[f, pl.ds(base, 16)]
            a = a + v * v
        sq_v[pl.ds(base, 16)] = a
        return 0
    lax.fori_loop(0, g_hi, body, 0)


def _topk_maxbm(i_pos, bo, n, g_hi, hT_v, sq_v, Bm_v, idx_scr):
    hs = [hT_v[f, i_pos] for f in range(H)]
    sq_i = sq_v[i_pos]
    iota = lax.iota(jnp.int32, 16)

    def gbody(g, carry):
        rd, ri = carry
        base = g * 16
        t = hs[0] * hT_v[0, pl.ds(base, 16)]
        for f in range(1, H):
            t = t + hs[f] * hT_v[f, pl.ds(base, 16)]
        d = sq_i + sq_v[pl.ds(base, 16)] - 2.0 * t
        pos_v = base + iota
        valid = (pos_v >= bo) & (pos_v < bo + n)
        d = jnp.where(valid, d, jnp.float32(jnp.inf))
        sd, si = plsc.sort_key_val(d, pos_v)
        rsd = lax.rev(sd, (0,))
        rsi = lax.rev(si, (0,))
        take = rsd < rd
        nd = jnp.where(take, rsd, rd)
        ni = jnp.where(take, rsi, ri)
        rd, ri = plsc.sort_key_val(nd, ni)
        return rd, ri

    rd0 = jnp.full((16,), jnp.inf, jnp.float32)
    ri0 = jnp.zeros((16,), jnp.int32)
    _, ri = lax.fori_loop(0, g_hi, gbody, (rd0, ri0))

    idx_scr[...] = ri
    m = Bm_v[idx_scr[0], :]
    for t in range(1, K):
        m = jnp.maximum(m, Bm_v[idx_scr[t], :])
    return m


def _sc_conv1_body(hT_hbm, C1_hbm, Bm1_hbm, seg_hbm, cnt_hbm,
                   Wd_hbm, Wcb_hbm, bc_hbm,
                   f1_hbm, f1T_hbm, C2_hbm, Bm2_hbm,
                   hT_v, C_v, Bm_v, sq_v, f1_v, f1T_v, C2_v, Bm2_v,
                   seg_v, cnt_v, Wd_v, Wcb_v, bc_v, idx_scr, row_scr):
    wid = lax.axis_index("s") * 2 + lax.axis_index("c")
    pltpu.sync_copy(seg_hbm, seg_v)
    pltpu.sync_copy(cnt_hbm, cnt_v)
    pltpu.sync_copy(Wd_hbm, Wd_v)
    pltpu.sync_copy(Wcb_hbm, Wcb_v)
    pltpu.sync_copy(bc_hbm, bc_v)
    bcv = bc_v[0, :]
    iota = lax.iota(jnp.int32, 16)

    for rep in range(2):
        s = wid * 2 + rep
        st = seg_v[s]
        s8 = (st // 8) * 8
        bo = st - s8
        n = cnt_v[s]
        g_hi = (bo + n + 15) // 16

        pltpu.sync_copy(hT_hbm.at[:, pl.ds(s8, NB)], hT_v)
        pltpu.sync_copy(C1_hbm.at[pl.ds(s8, NB)], C_v)
        pltpu.sync_copy(Bm1_hbm.at[pl.ds(s8, NB)], Bm_v)

        _seg_sq(hT_v, sq_v, g_hi)

        def nbody(i, _):
            pos = bo + i
            m = _topk_maxbm(pos, bo, n, g_hi, hT_v, sq_v, Bm_v, idx_scr)
            f1i = _elu(C_v[pos, :] + m)
            f1_v[i, :] = f1i
            plsc.store_scatter(f1T_v, [iota, jnp.zeros((16,), jnp.int32) + i],
                               f1i)
            row_scr[...] = f1i
            c2 = row_scr[0] * Wd_v[0, :]
            b2m = row_scr[0] * Wcb_v[0, :]
            for f in range(1, H):
                fs = row_scr[f]
                c2 = c2 + fs * Wd_v[f, :]
                b2m = b2m + fs * Wcb_v[f, :]
            C2_v[i, :] = c2 + bcv
            Bm2_v[i, :] = b2m
            return 0

        lax.fori_loop(0, n, nbody, 0)

        pltpu.sync_copy(f1_v, f1_hbm.at[s])
        pltpu.sync_copy(f1T_v, f1T_hbm.at[s])
        pltpu.sync_copy(C2_v, C2_hbm.at[s])
        pltpu.sync_copy(Bm2_v, Bm2_hbm.at[s])


def _sc_conv2_body(f1T_hbm, C2_hbm, Bm2_hbm, cnt_hbm,
                   Wo1_hbm, bo1_hbm, Wo2_hbm, bo2_hbm, Wo3_hbm, bo3_hbm,
                   out_hbm,
                   hT_v, C_v, Bm_v, sq_v, cnt_v,
                   Wo1_v, bo1_v, Wo2_v, bo2_v, Wo3_v, bo3_v,
                   idx_scr, row_scr):
    wid = lax.axis_index("s") * 2 + lax.axis_index("c")
    pltpu.sync_copy(cnt_hbm, cnt_v)
    pltpu.sync_copy(Wo1_hbm, Wo1_v)
    pltpu.sync_copy(bo1_hbm, bo1_v)
    pltpu.sync_copy(Wo2_hbm, Wo2_v)
    pltpu.sync_copy(bo2_hbm, bo2_v)
    pltpu.sync_copy(Wo3_hbm, Wo3_v)
    pltpu.sync_copy(bo3_hbm, bo3_v)

    for rep in range(2):
        s = wid * 2 + rep
        n = cnt_v[s]
        g_hi = (n + 15) // 16

        pltpu.sync_copy(f1T_hbm.at[s], hT_v)
        pltpu.sync_copy(C2_hbm.at[s], C_v)
        pltpu.sync_copy(Bm2_hbm.at[s], Bm_v)

        _seg_sq(hT_v, sq_v, g_hi)

        def nbody(i, acc):
            m = _topk_maxbm(i, 0, n, g_hi, hT_v, sq_v, Bm_v, idx_scr)
            f2i = _elu(C_v[i, :] + m)
            return acc + f2i

        acc = lax.fori_loop(0, n, nbody, jnp.zeros((16,), jnp.float32))
        nf = jnp.maximum(lax.convert_element_type(n, jnp.float32), 1.0)
        mean = acc / nf

        row_scr[...] = mean
        o1 = row_scr[0] * Wo1_v[0, :]
        for f in range(1, H):
            o1 = o1 + row_scr[f] * Wo1_v[f, :]
        o1 = _elu(o1 + bo1_v[0, :])
        row_scr[...] = o1
        o2 = row_scr[0] * Wo2_v[0, :]
        for f in range(1, 8):
            o2 = o2 + row_scr[f] * Wo2_v[f, :]
        o2 = _elu(o2 + bo2_v[0, :])
        row_scr[...] = o2
        o3 = row_scr[0] * Wo3_v[0, :]
        for f in range(1, 4):
            o3 = o3 + row_scr[f] * Wo3_v[f, :]
        o3 = o3 + bo3_v[0, :]
        row_scr[...] = o3
        pltpu.sync_copy(row_scr, out_hbm.at[s])


def _sc_conv1(hT, C1, Bm1, seg, cnt, Wd, Wcb, bc):
    mesh = plsc.VectorSubcoreMesh(core_axis_name="c", subcore_axis_name="s")
    f = pl.kernel(
        _sc_conv1_body,
        out_type=[
            jax.ShapeDtypeStruct((B, NB, H), jnp.float32),
            jax.ShapeDtypeStruct((B, H, NB), jnp.float32),
            jax.ShapeDtypeStruct((B, NB, H), jnp.float32),
            jax.ShapeDtypeStruct((B, NB, H), jnp.float32),
        ],
        mesh=mesh,
        scratch_types=[
            pltpu.VMEM((H, NB), jnp.float32),
            pltpu.VMEM((NB, H), jnp.float32),
            pltpu.VMEM((NB, H), jnp.float32),
            pltpu.VMEM((NB,), jnp.float32),
            pltpu.VMEM((NB, H), jnp.float32),
            pltpu.VMEM((H, NB), jnp.float32),
            pltpu.VMEM((NB, H), jnp.float32),
            pltpu.VMEM((NB, H), jnp.float32),
            pltpu.VMEM((B,), jnp.int32),
            pltpu.VMEM((B,), jnp.int32),
            pltpu.VMEM((H, H), jnp.float32),
            pltpu.VMEM((H, H), jnp.float32),
            pltpu.VMEM((1, H), jnp.float32),
            pltpu.VMEM((16,), jnp.int32),
            pltpu.VMEM((16,), jnp.float32),
        ],
    )
    return f(hT, C1, Bm1, seg, cnt, Wd, Wcb, bc)


def _sc_conv2(f1T, C2, Bm2, cnt, Wo1p, bo1p, Wo2p, bo2p, Wo3p, bo3p):
    mesh = plsc.VectorSubcoreMesh(core_axis_name="c", subcore_axis_name="s")
    f = pl.kernel(
        _sc_conv2_body,
        out_type=jax.ShapeDtypeStruct((B, H), jnp.float32),
        mesh=mesh,
        scratch_types=[
            pltpu.VMEM((H, NB), jnp.float32),
            pltpu.VMEM((NB, H), jnp.float32),
            pltpu.VMEM((NB, H), jnp.float32),
            pltpu.VMEM((NB,), jnp.float32),
            pltpu.VMEM((B,), jnp.int32),
            pltpu.VMEM((H, H), jnp.float32),
            pltpu.VMEM((1, H), jnp.float32),
            pltpu.VMEM((8, H), jnp.float32),
            pltpu.VMEM((1, H), jnp.float32),
            pltpu.VMEM((4, H), jnp.float32),
            pltpu.VMEM((1, H), jnp.float32),
            pltpu.VMEM((16,), jnp.int32),
            pltpu.VMEM((16,), jnp.float32),
        ],
    )
    return f(f1T, C2, Bm2, cnt, Wo1p, bo1p, Wo2p, bo2p, Wo3p, bo3p)


def kernel(x_pf, batch_pf, W1, b1, W2, b2, Wc, bc, Wo1, bo1, Wo2, bo2,
           Wo3, bo3):
    seg_ids = jnp.arange(B, dtype=batch_pf.dtype)
    seg_start = jnp.searchsorted(batch_pf, seg_ids, side="left").astype(jnp.int32)
    seg_end = jnp.searchsorted(batch_pf, seg_ids, side="right").astype(jnp.int32)
    cnt = (seg_end - seg_start).astype(jnp.int32)

    Wct, Wcb = Wc[:H], Wc[H:]
    Wd = Wct - Wcb

    h, C1, Bm1 = _tc_encoder(
        x_pf, W1, b1.reshape(1, H), W2, b2.reshape(1, H),
        Wd, bc.reshape(1, H), Wcb)

    hT = jnp.pad(h.T, ((0, 0), (0, NTOT - N)))
    C1p = jnp.pad(C1, ((0, NTOT - N), (0, 0)))
    Bm1p = jnp.pad(Bm1, ((0, NTOT - N), (0, 0)))

    f1, f1T, C2, Bm2 = _sc_conv1(hT, C1p, Bm1p, seg_start, cnt, Wd, Wcb,
                                 bc.reshape(1, H))
    del f1

    Wo1p = jnp.pad(Wo1, ((0, 0), (0, H - 8)))
    bo1p = jnp.pad(bo1, (0, H - 8)).reshape(1, H)
    Wo2p = jnp.pad(Wo2, ((0, 0), (0, H - 4)))
    bo2p = jnp.pad(bo2, (0, H - 4)).reshape(1, H)
    Wo3p = jnp.pad(Wo3, ((0, 0), (0, H - 1)))
    bo3p = jnp.pad(bo3, (0, H - 1)).reshape(1, H)

    heads = _sc_conv2(f1T, C2, Bm2, cnt, Wo1p, bo1p, Wo2p, bo2p, Wo3p, bo3p)
    o = heads[:, :1]
    return (o, jnp.arange(B, dtype=jnp.int32))


# speedup vs baseline: 32.6328x; 32.6328x over previous
import jax
import jax.numpy as jnp
from jax import lax
from jax.experimental import pallas as pl
from jax.experimental.pallas import tpu as pltpu
from jax.experimental.pallas import tpu_sc as plsc

N = 10000
B = 64
K = 8
H = 16
NBR = 640
W16 = 768
NTOT = N + W16
BP = B + 16


def _elu(x):
    c = [1.0, 0.5, 1 / 6., 1 / 24., 1 / 120., 1 / 720., 1 / 5040., 1 / 40320.]
    p = jnp.float32(c[-1])
    for k in reversed(c[:-1]):
        p = jnp.float32(k) + x * p
    p = x * p
    em = jnp.where(jnp.abs(x) < 0.5, p, jnp.exp(x) - 1.0)
    return jnp.where(x > 0, x, em)


def _round_bf16(v):
    u = plsc.bitcast(v, jnp.uint32)
    r = (u + jnp.uint32(0x7FFF) + ((u >> 16) & jnp.uint32(1))) \
        & jnp.uint32(0xFFFF0000)
    return plsc.bitcast(r, jnp.float32)


def _dyn_lane(v, r):
    idx = jnp.zeros((16,), jnp.int32) + r
    return v.at[idx].get(mode="promise_in_bounds")[0]


def _sload(ref, idx):
    ga = pl.multiple_of((idx // 16) * 16, 16)
    return _dyn_lane(ref[pl.ds(ga, 16)], idx - ga)


def _rb(x):
    return x.astype(jnp.bfloat16).astype(jnp.float32)


def _dot(a, b):
    return jnp.dot(a, b, preferred_element_type=jnp.float32,
                   precision=lax.Precision.HIGHEST)


def _tc_encoder_body(x_ref, W1_ref, b1_ref, W2_ref, b2_ref, Wct_ref,
                     h_ref, hb_ref, C1_ref):
    x = x_ref[...]
    h1 = _elu(_dot(_rb(x), W1_ref[...]) + b1_ref[...])
    h = _elu(_dot(_rb(h1), W2_ref[...]) + b2_ref[...])
    hb = _rb(h)
    h_ref[...] = h
    hb_ref[...] = hb
    C1_ref[...] = _dot(hb, Wct_ref[...])


def _tc_encoder(x, W1, b1, W2, b2, Wct):
    R = 1000
    grid = (N // R,)
    full = lambda shape: pl.BlockSpec(shape, lambda i: (0, 0))
    return pl.pallas_call(
        _tc_encoder_body,
        grid=grid,
        in_specs=[
            pl.BlockSpec((R, 4), lambda i: (i, 0)),
            full((4, H)), full((1, H)), full((H, H)), full((1, H)),
            full((H, H)),
        ],
        out_specs=[
            pl.BlockSpec((R, H), lambda i: (i, 0)),
            pl.BlockSpec((R, H), lambda i: (i, 0)),
            pl.BlockSpec((R, H), lambda i: (i, 0)),
        ],
        out_shape=[jax.ShapeDtypeStruct((N, H), jnp.float32)] * 3,
    )(x, W1, b1, W2, b2, Wct)


def _tree_sum(terms):
    lvl = list(terms)
    while len(lvl) > 1:
        lvl = [lvl[i] + lvl[i + 1] for i in range(0, len(lvl), 2)]
    return lvl[0]


def _topk(i_pos, bo, n, g_hi, hb_v, sq_v):
    ga = pl.multiple_of((i_pos // 16) * 16, 16)
    r = i_pos - ga
    hs = [_dyn_lane(hb_v[f, pl.ds(ga, 16)], r) for f in range(H)]
    sq_i = _dyn_lane(sq_v[0, pl.ds(ga, 16)], r)
    iota = lax.iota(jnp.int32, 16)

    def gbody(g, carry):
        rd, ri = carry
        base = pl.multiple_of(g * 16, 16)
        t = _tree_sum([hs[f] * hb_v[f, pl.ds(base, 16)] for f in range(H)])
        d = (sq_i - 2.0 * t) + sq_v[0, pl.ds(base, 16)]
        pos_v = base + iota
        valid = (pos_v >= bo) & (pos_v < bo + n)
        d = jnp.where(valid, d, jnp.float32(jnp.inf))
        sd, si = plsc.sort_key_val(d, pos_v)
        rsd = lax.rev(sd, (0,))
        rsi = lax.rev(si, (0,))
        take = rsd < rd
        nd = jnp.where(take, rsd, rd)
        ni = jnp.where(take, rsi, ri)
        rd2, ri2 = plsc.sort_key_val(nd, ni)
        return rd2, ri2

    rd0 = jnp.full((16,), jnp.inf, jnp.float32)
    ri0 = jnp.zeros((16,), jnp.int32)
    _, ri = lax.fori_loop(0, g_hi, gbody, (rd0, ri0))
    return ri


def _edge_max(ri, hrow_i, h_v, Wcb_v):
    m = None
    for t in range(K):
        diff = _round_bf16(h_v[ri[t], :] - hrow_i)
        e = _tree_sum([diff[f] * Wcb_v[f, :] for f in range(H)])
        m = e if m is None else jnp.maximum(m, e)
    return m


def _sc_conv1_body(hbT_hbm, sqT_hbm, C1_hbm, h_hbm, seg_hbm, cnt_hbm,
                   Wct_hbm, Wcb_hbm, bc_hbm,
                   f1_hbm, f1bT_hbm, C2_hbm,
                   hb_v, sq_v, C_v, h_v, f1_v, f1bT_v, C2_v,
                   seg_v, cnt_v, Wct_v, Wcb_v, bc_v):
    wid = lax.axis_index("s") * 2 + lax.axis_index("c")
    pltpu.sync_copy(seg_hbm, seg_v)
    pltpu.sync_copy(cnt_hbm, cnt_v)
    pltpu.sync_copy(Wct_hbm, Wct_v)
    pltpu.sync_copy(Wcb_hbm, Wcb_v)
    pltpu.sync_copy(bc_hbm, bc_v)
    bcv = bc_v[0, :]
    iota = lax.iota(jnp.int32, 16)
    lane0 = iota == 0
    zeros = jnp.zeros((16,), jnp.int32)

    for rep in range(2):
        s = wid * 2 + rep
        st = _sload(seg_v, s)
        s128 = (st // 128) * 128
        bo = st - s128
        n = _sload(cnt_v, s)
        g_hi = (bo + n + 15) // 16

        pltpu.sync_copy(hbT_hbm.at[:, pl.ds(s128, W16)], hb_v)
        pltpu.sync_copy(sqT_hbm.at[:, pl.ds(s128, W16)], sq_v)
        pltpu.sync_copy(C1_hbm.at[pl.ds(s128, NBR)], C_v)
        pltpu.sync_copy(h_hbm.at[pl.ds(s128, NBR)], h_v)

        def nbody(i, _):
            pos = bo + i
            ri = _topk(pos, bo, n, g_hi, hb_v, sq_v)
            m = _edge_max(ri, h_v[pos, :], h_v, Wcb_v)
            f1i = _elu((C_v[pos, :] + m) + bcv)
            f1b = _round_bf16(f1i)
            f1_v[i, :] = f1i
            col = zeros + i
            plsc.store_scatter(f1bT_v, [iota, col], f1b)
            C2_v[i, :] = _tree_sum([f1b[f] * Wct_v[f, :] for f in range(H)])
            return 0

        lax.fori_loop(0, n, nbody, 0)

        pltpu.sync_copy(f1_v, f1_hbm.at[s])
        pltpu.sync_copy(f1bT_v, f1bT_hbm.at[s])
        pltpu.sync_copy(C2_v, C2_hbm.at[s])


def _sc_conv2_body(f1_hbm, f1bT_hbm, sq2T_hbm, C2_hbm, cnt_hbm, Wcb_hbm,
                   bc_hbm, Wo1_hbm, bo1_hbm, Wo2_hbm, bo2_hbm, Wo3_hbm,
                   bo3_hbm, out_hbm,
                   h_v, hb_v, sq_v, C_v, cnt_v, Wcb_v, bc_v,
                   Wo1_v, bo1_v, Wo2_v, bo2_v, Wo3_v, bo3_v,
                   row_scr):
    wid = lax.axis_index("s") * 2 + lax.axis_index("c")
    pltpu.sync_copy(cnt_hbm, cnt_v)
    pltpu.sync_copy(Wcb_hbm, Wcb_v)
    pltpu.sync_copy(bc_hbm, bc_v)
    bcv = bc_v[0, :]
    pltpu.sync_copy(Wo1_hbm, Wo1_v)
    pltpu.sync_copy(bo1_hbm, bo1_v)
    pltpu.sync_copy(Wo2_hbm, Wo2_v)
    pltpu.sync_copy(bo2_hbm, bo2_v)
    pltpu.sync_copy(Wo3_hbm, Wo3_v)
    pltpu.sync_copy(bo3_hbm, bo3_v)

    for rep in range(2):
        s = wid * 2 + rep
        n = _sload(cnt_v, s)
        g_hi = (n + 15) // 16

        pltpu.sync_copy(f1_hbm.at[s], h_v)
        pltpu.sync_copy(f1bT_hbm.at[s], hb_v)
        pltpu.sync_copy(sq2T_hbm.at[s], sq_v)
        pltpu.sync_copy(C2_hbm.at[s], C_v)

        def nbody(i, acc):
            ri = _topk(i, 0, n, g_hi, hb_v, sq_v)
            m = _edge_max(ri, h_v[i, :], h_v, Wcb_v)
            f2i = _elu((C_v[i, :] + m) + bcv)
            return acc + f2i

        acc = lax.fori_loop(0, n, nbody, jnp.zeros((16,), jnp.float32))
        nf = jnp.maximum(lax.convert_element_type(n, jnp.float32), 1.0)
        mean = _round_bf16(acc / nf)

        o1 = mean[0] * Wo1_v[0, :]
        for f in range(1, H):
            o1 = o1 + mean[f] * Wo1_v[f, :]
        o1 = _round_bf16(_elu(o1 + bo1_v[0, :]))
        o2 = o1[0] * Wo2_v[0, :]
        for f in range(1, 8):
            o2 = o2 + o1[f] * Wo2_v[f, :]
        o2 = _round_bf16(_elu(o2 + bo2_v[0, :]))
        o3 = o2[0] * Wo3_v[0, :]
        for f in range(1, 4):
            o3 = o3 + o2[f] * Wo3_v[f, :]
        o3 = o3 + bo3_v[0, :]
        row_scr[...] = o3
        pltpu.sync_copy(row_scr, out_hbm.at[s])


_SC_PARAMS = dict(
    compiler_params=pltpu.CompilerParams(needs_layout_passes=False,
                                         use_tc_tiling_on_sc=False),
)


def _sc_conv1(hbT, sqT, C1, hp, seg, cnt, Wct, Wcb, bc):
    mesh = plsc.VectorSubcoreMesh(core_axis_name="c", subcore_axis_name="s")
    f = pl.kernel(
        _sc_conv1_body,
        out_type=[
            jax.ShapeDtypeStruct((B, NBR, H), jnp.float32),
            jax.ShapeDtypeStruct((B, H, W16), jnp.float32),
            jax.ShapeDtypeStruct((B, NBR, H), jnp.float32),
        ],
        mesh=mesh,
        scratch_types=[
            pltpu.VMEM((H, W16), jnp.float32),
            pltpu.VMEM((1, W16), jnp.float32),
            pltpu.VMEM((NBR, H), jnp.float32),
            pltpu.VMEM((NBR, H), jnp.float32),
            pltpu.VMEM((NBR, H), jnp.float32),
            pltpu.VMEM((H, W16), jnp.float32),
            pltpu.VMEM((NBR, H), jnp.float32),
            pltpu.VMEM((BP,), jnp.int32),
            pltpu.VMEM((BP,), jnp.int32),
            pltpu.VMEM((H, H), jnp.float32),
            pltpu.VMEM((H, H), jnp.float32),
            pltpu.VMEM((1, H), jnp.float32),
        ],
        **_SC_PARAMS,
    )
    return f(hbT, sqT, C1, hp, seg, cnt, Wct, Wcb, bc)


def _sc_conv2(f1, f1bT, sq2T, C2, cnt, Wcb, bc, Wo1p, bo1p, Wo2p, bo2p,
              Wo3p, bo3p):
    mesh = plsc.VectorSubcoreMesh(core_axis_name="c", subcore_axis_name="s")
    f = pl.kernel(
        _sc_conv2_body,
        out_type=jax.ShapeDtypeStruct((B, H), jnp.float32),
        mesh=mesh,
        scratch_types=[
            pltpu.VMEM((NBR, H), jnp.float32),
            pltpu.VMEM((H, W16), jnp.float32),
            pltpu.VMEM((1, NBR), jnp.float32),
            pltpu.VMEM((NBR, H), jnp.float32),
            pltpu.VMEM((BP,), jnp.int32),
            pltpu.VMEM((H, H), jnp.float32),
            pltpu.VMEM((1, H), jnp.float32),
            pltpu.VMEM((H, H), jnp.float32),
            pltpu.VMEM((1, H), jnp.float32),
            pltpu.VMEM((8, H), jnp.float32),
            pltpu.VMEM((1, H), jnp.float32),
            pltpu.VMEM((4, H), jnp.float32),
            pltpu.VMEM((1, H), jnp.float32),
            pltpu.VMEM((16,), jnp.float32),
        ],
        **_SC_PARAMS,
    )
    return f(f1, f1bT, sq2T, C2, cnt, Wcb, bc, Wo1p, bo1p, Wo2p, bo2p,
             Wo3p, bo3p)


def kernel(x_pf, batch_pf, W1, b1, W2, b2, Wc, bc, Wo1, bo1, Wo2, bo2,
           Wo3, bo3):
    seg_ids = jnp.arange(B, dtype=batch_pf.dtype)
    seg_start = jnp.searchsorted(batch_pf, seg_ids, side="left").astype(jnp.int32)
    seg_end = jnp.searchsorted(batch_pf, seg_ids, side="right").astype(jnp.int32)
    cnt = (seg_end - seg_start).astype(jnp.int32)
    seg_start_p = jnp.pad(seg_start, (0, BP - B))
    cnt_p = jnp.pad(cnt, (0, BP - B))

    rb = lambda w: w.astype(jnp.bfloat16).astype(jnp.float32)
    Wct_b = rb(Wc[:H])
    Wcb_b = rb(Wc[H:])

    h, hb, C1 = _tc_encoder(
        x_pf, rb(W1), b1.reshape(1, H), rb(W2), b2.reshape(1, H), Wct_b)

    sq = jnp.sum(h * h, axis=1)
    hbT = jnp.pad(hb.T, ((0, 0), (0, NTOT - N)))
    sqT = jnp.pad(sq[None, :], ((0, 0), (0, NTOT - N)))
    C1p = jnp.pad(C1, ((0, NTOT - N), (0, 0)))
    hp = jnp.pad(h, ((0, NTOT - N), (0, 0)))

    f1, f1bT, C2 = _sc_conv1(hbT, sqT, C1p, hp, seg_start_p, cnt_p,
                             Wct_b, Wcb_b, bc.reshape(1, H))
    sq2T = jnp.sum(f1 * f1, axis=-1)[:, None, :]

    Wo1p = jnp.pad(rb(Wo1), ((0, 0), (0, H - 8)))
    bo1p = jnp.pad(bo1, (0, H - 8)).reshape(1, H)
    Wo2p = jnp.pad(rb(Wo2), ((0, 0), (0, H - 4)))
    bo2p = jnp.pad(bo2, (0, H - 4)).reshape(1, H)
    Wo3p = jnp.pad(rb(Wo3), ((0, 0), (0, H - 1)))
    bo3p = jnp.pad(bo3, (0, H - 1)).reshape(1, H)

    heads = _sc_conv2(f1, f1bT, sq2T, C2, cnt_p, Wcb_b, bc.reshape(1, H),
                      Wo1p, bo1p, Wo2p, bo2p, Wo3p, bo3p)
    o = heads[:, :1]
    return (o, jnp.arange(B, dtype=jnp.int32))
